# Initial kernel scaffold; baseline (speedup 1.0000x reference)
import jax
import jax.numpy as jnp
from jax.experimental import pallas as pl


def kernel(feature, edge_index, W, idx):
    raise NotImplementedError("write your pallas kernel here")


# speedup vs baseline: 3.9720x; 3.9720x over previous
import functools

import jax
import jax.numpy as jnp
from jax import lax
from jax.experimental import pallas as pl
from jax.experimental.pallas import tpu as pltpu
from jax.experimental.pallas import tpu_sc as plsc

N = 10000
E = 320000
D = 128
D_OUT = 128
B = 1024

NC = 2
NS = 16
CHUNK = 128
N_PAD = 10240
EPT = (E + NS * CHUNK - 1) // (NS * CHUNK) * CHUNK
E_PAD = EPT * NS
NCHUNK = EPT // CHUNK
B_PT = B // NS


def _sc_agg_body(feat_hbm, src_hbm, dst_hbm, zero_hbm, ones_hbm,
                 acc_hbm, deg_hbm,
                 u_sh, src_v, dst_v, rows_v, sem):
    cid = lax.axis_index("c")
    sid = lax.axis_index("s")

    rpt = N_PAD // NS
    pltpu.sync_copy(zero_hbm.at[pl.ds(sid * rpt, rpt)],
                    u_sh.at[pl.ds(sid * rpt, rpt)])
    plsc.subcore_barrier()

    @pl.when(cid == 0)
    def _acc_core():
        def chunk_body(c, carry):
            pltpu.sync_copy(src_hbm.at[sid, c], src_v)
            pltpu.sync_copy(dst_hbm.at[sid, c], dst_v)
            pltpu.async_copy(feat_hbm.at[src_v], rows_v, sem).wait()
            pltpu.sync_copy(rows_v, u_sh.at[dst_v], add=True)
            return carry

        lax.fori_loop(0, NCHUNK, chunk_body, 0)

    @pl.when(cid == 1)
    def _deg_core():
        pltpu.sync_copy(ones_hbm, rows_v)

        def chunk_body(c, carry):
            pltpu.sync_copy(dst_hbm.at[sid, c], dst_v)
            pltpu.sync_copy(rows_v, u_sh.at[dst_v], add=True)
            return carry

        lax.fori_loop(0, NCHUNK, chunk_body, 0)

    plsc.subcore_barrier()

    @pl.when(cid == 0)
    def _acc_out():
        pltpu.sync_copy(u_sh.at[pl.ds(sid * rpt, rpt)],
                        acc_hbm.at[pl.ds(sid * rpt, rpt)])

    @pl.when(cid == 1)
    def _deg_out():
        pltpu.sync_copy(u_sh.at[pl.ds(sid * rpt, rpt)],
                        deg_hbm.at[pl.ds(sid * rpt, rpt)])


_sc_agg = functools.partial(
    pl.kernel,
    out_type=[
        jax.ShapeDtypeStruct((N_PAD, D), jnp.float32),
        jax.ShapeDtypeStruct((N_PAD, 128), jnp.float32),
    ],
    mesh=plsc.VectorSubcoreMesh(core_axis_name="c", subcore_axis_name="s"),
    scratch_types=[
        pltpu.VMEM_SHARED((N_PAD, D), jnp.float32),
        pltpu.VMEM((CHUNK,), jnp.int32),
        pltpu.VMEM((CHUNK,), jnp.int32),
        pltpu.VMEM((CHUNK, D), jnp.float32),
        pltpu.SemaphoreType.DMA,
    ],
)(_sc_agg_body)


def _sc_sel_body(acc_hbm, deg_hbm, idx_hbm, p_hbm, dg_hbm,
                 idxs_v, orow_v, sem):
    cid = lax.axis_index("c")
    sid = lax.axis_index("s")
    pltpu.sync_copy(idx_hbm.at[pl.ds(sid * B_PT, B_PT)], idxs_v)

    @pl.when(cid == 0)
    def _sel_acc():
        pltpu.async_copy(acc_hbm.at[idxs_v], orow_v, sem).wait()
        pltpu.sync_copy(orow_v, p_hbm.at[pl.ds(sid * B_PT, B_PT)])

    @pl.when(cid == 1)
    def _sel_deg():
        pltpu.async_copy(deg_hbm.at[idxs_v], orow_v, sem).wait()
        pltpu.sync_copy(orow_v, dg_hbm.at[pl.ds(sid * B_PT, B_PT)])


_sc_sel = functools.partial(
    pl.kernel,
    out_type=[
        jax.ShapeDtypeStruct((B, D), jnp.float32),
        jax.ShapeDtypeStruct((B, 128), jnp.float32),
    ],
    mesh=plsc.VectorSubcoreMesh(core_axis_name="c", subcore_axis_name="s"),
    scratch_types=[
        pltpu.VMEM((B_PT,), jnp.int32),
        pltpu.VMEM((B_PT, D), jnp.float32),
        pltpu.SemaphoreType.DMA,
    ],
)(_sc_sel_body)


def _tc_body(p_ref, dg_ref, w_ref, o_ref):
    d = dg_ref[:, 0]
    h = jnp.dot(p_ref[...], w_ref[...], preferred_element_type=jnp.float32)
    o_ref[...] = h / jnp.clip(d, 1.0)[:, None]


def kernel(feature, edge_index, W, idx):
    src = edge_index[0].astype(jnp.int32)
    dst = edge_index[1].astype(jnp.int32)
    pad = E_PAD - E
    src_p = jnp.concatenate([src, jnp.zeros((pad,), jnp.int32)])
    dst_p = jnp.concatenate([dst, jnp.full((pad,), N, jnp.int32)])
    src3 = src_p.reshape(NS, NCHUNK, CHUNK)
    dst3 = dst_p.reshape(NS, NCHUNK, CHUNK)
    zero = jnp.zeros((N_PAD, D), jnp.float32)
    ones = jnp.ones((CHUNK, 128), jnp.float32)

    acc, deg = _sc_agg(feature, src3, dst3, zero, ones)
    p, dg = _sc_sel(acc, deg, idx.astype(jnp.int32))

    return pl.pallas_call(
        _tc_body,
        out_shape=jax.ShapeDtypeStruct((B, D_OUT), jnp.float32),
    )(p, dg, W)
